# baseline (speedup 1.0000x reference)
import jax
import jax.numpy as jnp
from jax.experimental import pallas as pl

NODE = 128
FEAT = 16
ROW_TILE = 480


def _tc_kernel(x_ref, e_ref, o_ref):
    x = x_ref[...]
    y = x * e_ref[...]
    z = jnp.sum(y.reshape(x.shape[0], NODE, FEAT), axis=-1)
    o_ref[...] = z


def kernel(input_tensor, embedding):
    b, s, h, d = input_tensor.shape
    rows = b * s * h
    x2 = input_tensor.reshape(rows, d)
    e2 = embedding.reshape(1, d)
    grid = rows // ROW_TILE
    out = pl.pallas_call(
        _tc_kernel,
        grid=(grid,),
        in_specs=[
            pl.BlockSpec((ROW_TILE, d), lambda i: (i, 0)),
            pl.BlockSpec((1, d), lambda i: (0, 0)),
        ],
        out_specs=pl.BlockSpec((ROW_TILE, NODE), lambda i: (i, 0)),
        out_shape=jax.ShapeDtypeStruct((rows, NODE), jnp.float32),
    )(x2, e2)
    return out.reshape(b, s, h, NODE)


# speedup vs baseline: 8.7747x; 8.7747x over previous
import jax
import jax.numpy as jnp
from jax.experimental import pallas as pl

NODE = 128
FEAT = 16
ROW_TILE = 480


def _tc_kernel(x_ref, w_ref, o_ref):
    o_ref[...] = jnp.dot(x_ref[...], w_ref[...],
                         preferred_element_type=jnp.float32)


def kernel(input_tensor, embedding):
    b, s, h, d = input_tensor.shape
    rows = b * s * h
    x2 = input_tensor.reshape(rows, d)
    k = jnp.arange(d)
    w = jnp.zeros((d, NODE), jnp.float32).at[k, k // FEAT].set(
        embedding.reshape(d))
    grid = rows // ROW_TILE
    out = pl.pallas_call(
        _tc_kernel,
        grid=(grid,),
        in_specs=[
            pl.BlockSpec((ROW_TILE, d), lambda i: (i, 0)),
            pl.BlockSpec((d, NODE), lambda i: (0, 0)),
        ],
        out_specs=pl.BlockSpec((ROW_TILE, NODE), lambda i: (i, 0)),
        out_shape=jax.ShapeDtypeStruct((rows, NODE), jnp.float32),
    )(x2, w)
    return out.reshape(b, s, h, NODE)


# speedup vs baseline: 10.3730x; 1.1821x over previous
import jax
import jax.numpy as jnp
from jax.experimental import pallas as pl

NODE = 128
FEAT = 16
ROW_TILE = 960


def _tc_kernel(x_ref, w_ref, o_ref):
    o_ref[...] = jnp.dot(x_ref[...], w_ref[...],
                         preferred_element_type=jnp.float32)


def kernel(input_tensor, embedding):
    b, s, h, d = input_tensor.shape
    rows = b * s * h
    x2 = input_tensor.reshape(rows, d)
    k = jnp.arange(d)
    w = jnp.zeros((d, NODE), jnp.float32).at[k, k // FEAT].set(
        embedding.reshape(d))
    grid = rows // ROW_TILE
    out = pl.pallas_call(
        _tc_kernel,
        grid=(grid,),
        in_specs=[
            pl.BlockSpec((ROW_TILE, d), lambda i: (i, 0)),
            pl.BlockSpec((d, NODE), lambda i: (0, 0)),
        ],
        out_specs=pl.BlockSpec((ROW_TILE, NODE), lambda i: (i, 0)),
        out_shape=jax.ShapeDtypeStruct((rows, NODE), jnp.float32),
    )(x2, w)
    return out.reshape(b, s, h, NODE)
